# Initial kernel scaffold; baseline (speedup 1.0000x reference)
import jax
import jax.numpy as jnp
from jax.experimental import pallas as pl


def kernel(h_q_raw, h_m_raw, edge_index, Wq, bq, Wm, bm, W0, as0, ad0, b0, W1, as1, ad1, b1):
    raise NotImplementedError("write your pallas kernel here")


# speedup vs baseline: 22.0861x; 22.0861x over previous
import functools

import jax
import jax.numpy as jnp
from jax import lax
from jax.experimental import pallas as pl
from jax.experimental.pallas import tpu as pltpu
from jax.experimental.pallas import tpu_sc as plsc

NQ = 5000
NM = 5000
NN = NQ + NM
EE = 320000
DIN = 128
HID = 128
HEADS = 4

NC = 2
NS = 16
NW = NC * NS
EPT = EE // NW
CH = 40
NCHK = EPT // CH
NPAD = 10112
NPT = NPAD // NS
RW = 16

_mesh = plsc.VectorSubcoreMesh(core_axis_name="c", subcore_axis_name="s")


def _f32(*shape):
    return jax.ShapeDtypeStruct(shape, jnp.float32)


def _elu(v):
    return jnp.where(v > 0, v, jnp.exp(jnp.minimum(v, 0.0)) - 1.0)


def _proj_body(hq_ref, hm_ref, wq_ref, bq_ref, wm_ref, bm_ref, h_ref):
    hq = _elu(hq_ref[...] @ wq_ref[...] + bq_ref[...][None, :])
    hm = _elu(hm_ref[...] @ wm_ref[...] + bm_ref[...][None, :])
    pad = jnp.zeros((NPAD - NN, DIN), jnp.float32)
    h_ref[...] = jnp.concatenate([hq, hm, pad], axis=0)


def _proj(hq, hm, wq, bq, wm, bm):
    return pl.pallas_call(_proj_body, out_shape=_f32(NPAD, DIN))(hq, hm, wq, bq, wm, bm)


def _xal_body(li_ref, h_ref, w_ref, a_ref, x_ref, al_ref):
    li = li_ref[0]
    x = jnp.dot(h_ref[...], w_ref[li],
                preferred_element_type=jnp.float32)
    x_ref[...] = x
    al_ref[...] = jnp.dot(x, a_ref[li], preferred_element_type=jnp.float32)


def _xal(li, h, ws, ams):
    blk = NPAD // 8
    return pl.pallas_call(
        _xal_body,
        grid=(NPAD // blk,),
        in_specs=[
            pl.BlockSpec(memory_space=pltpu.SMEM),
            pl.BlockSpec((blk, HID), lambda i: (i, 0)),
            pl.BlockSpec((2, HID, HEADS * HID), lambda i: (0, 0, 0)),
            pl.BlockSpec((2, HEADS * HID, RW), lambda i: (0, 0, 0)),
        ],
        out_specs=[
            pl.BlockSpec((blk, HEADS * HID), lambda i: (i, 0)),
            pl.BlockSpec((blk, RW), lambda i: (i, 0)),
        ],
        out_shape=(_f32(NPAD, HEADS * HID), _f32(NPAD, RW)),
    )(li, h, ws, ams)


def _merge2(li, o0, o1, bs):
    def body(li_ref, o0_ref, o1_ref, b_ref, raw_ref, h_ref):
        v = o0_ref[...] + o1_ref[...] + b_ref[li_ref[0]][None, :]
        raw_ref[...] = v
        h_ref[...] = _elu(v)

    return pl.pallas_call(
        body,
        in_specs=[
            pl.BlockSpec(memory_space=pltpu.SMEM),
            pl.BlockSpec((NPAD, HID), lambda: (0, 0)),
            pl.BlockSpec((NPAD, HID), lambda: (0, 0)),
            pl.BlockSpec((2, HID), lambda: (0, 0)),
        ],
        out_shape=(_f32(NPAD, HID), _f32(NPAD, HID)))(li, o0, o1, bs)


def _split_body(raw_ref, q_ref, m_ref):
    q_ref[...] = raw_ref[...][:NQ]
    m_ref[...] = raw_ref[...][NQ:NN]


def _split(raw):
    return pl.pallas_call(
        _split_body, out_shape=(_f32(NQ, HID), _f32(NM, HID)))(raw)


def _inv_body(dp_ref, inv_ref):
    d = dp_ref[...][0] + dp_ref[...][1]
    i16 = 0.25 / (d + 1e-16)
    inv_ref[...] = jnp.concatenate([i16] * (HID // RW), axis=1)


def _inv(den_parts):
    return pl.pallas_call(_inv_body, out_shape=_f32(NPAD, HID))(den_parts)


ZRD = 8


def _pass1_body(alpha_hbm, edges_hbm, ex_hbm, den_hbm,
                alpha_sh, den_sh, zero_v, ed_ch,
                s_rows, d_rows, exf_v, ex2_v, sem_s, sem_d):
    cid = lax.axis_index("c")
    sid = lax.axis_index("s")
    wid = cid * NS + sid

    def _zrow(i, _):
        zero_v[i, pl.ds(0, RW)] = jnp.zeros((RW,), jnp.float32)
        return _
    lax.fori_loop(0, ZRD, _zrow, None)

    def _zden(r, _):
        pltpu.sync_copy(zero_v, den_sh.at[pl.ds(sid * NPT + r * ZRD, ZRD)])
        return _
    lax.fori_loop(0, NPT // ZRD, _zden, None)
    pltpu.sync_copy(alpha_hbm.at[pl.ds(sid * NPT, NPT)],
                    alpha_sh.at[pl.ds(sid * NPT, NPT)])
    plsc.subcore_barrier()
    lt = lax.iota(jnp.int32, 16)
    shfl = jnp.bitwise_and(lt, HEADS * 2 - 1) + HEADS * 2

    def _chunk(j, _):
        pltpu.sync_copy(edges_hbm.at[wid * NCHK + j], ed_ch)
        cs = pltpu.async_copy(alpha_sh.at[ed_ch.at[0]], s_rows, sem_s)
        cd = pltpu.async_copy(alpha_sh.at[ed_ch.at[1]], d_rows, sem_d)
        cs.wait()
        cd.wait()

        def _edge(e, _):
            dsh = d_rows[e, pl.ds(0, RW)].at[shfl].get(
                mode="promise_in_bounds")
            al = s_rows[e, pl.ds(0, RW)] + dsh
            al = jnp.maximum(al, 0.2 * al)
            ex = jnp.exp(al)
            exf_v[pl.ds(e * RW, RW)] = ex
            ex2_v[e, pl.ds(0, RW)] = ex
            return _
        lax.fori_loop(0, CH, _edge, None)

        pltpu.sync_copy(ex2_v, den_sh.at[ed_ch.at[1]], add=True)
        eoff = pl.multiple_of((wid * NCHK + j) * CH * RW, 8)
        pltpu.sync_copy(exf_v, ex_hbm.at[pl.ds(eoff, CH * RW)])
        return _

    lax.fori_loop(0, NCHK, _chunk, None)

    plsc.subcore_barrier()
    pltpu.sync_copy(den_sh.at[pl.ds(sid * NPT, NPT)],
                    den_hbm.at[cid, pl.ds(sid * NPT, NPT)])


_pass1 = functools.partial(
    pl.kernel,
    out_type=(jax.ShapeDtypeStruct((EE * RW,), jnp.float32),
              jax.ShapeDtypeStruct((NC, NPAD, RW), jnp.float32)),
    mesh=_mesh,
    scratch_types=[
        pltpu.VMEM_SHARED((NPAD, RW), jnp.float32),
        pltpu.VMEM_SHARED((NPAD, RW), jnp.float32),
        pltpu.VMEM((ZRD, RW), jnp.float32),
        pltpu.VMEM((2, CH), jnp.int32),
        pltpu.VMEM((CH, RW), jnp.float32),
        pltpu.VMEM((CH, RW), jnp.float32),
        pltpu.VMEM((CH * RW,), jnp.float32),
        pltpu.VMEM((CH, RW), jnp.float32),
        pltpu.SemaphoreType.DMA,
        pltpu.SemaphoreType.DMA,
    ],
)(_pass1_body)


ZR = 8


def _pass2_body(x_hbm, edges_hbm, ex_hbm, inv_hbm, out_hbm,
                out_sh, zero_v, ed_ch,
                exf_v, iv_v, rows_v, m_v, sem_i, sem_x, sem_e):
    cid = lax.axis_index("c")
    sid = lax.axis_index("s")
    wid = cid * NS + sid

    def _zrow(i, _):
        for k in range(HID // 16):
            zero_v[i, pl.ds(k * 16, 16)] = jnp.zeros((16,), jnp.float32)
        return _
    lax.fori_loop(0, ZR, _zrow, None)

    def _zout(r, _):
        pltpu.sync_copy(zero_v, out_sh.at[pl.ds(sid * NPT + r * ZR, ZR)])
        return _
    lax.fori_loop(0, NPT // ZR, _zout, None)

    plsc.subcore_barrier()

    def _chunk(j, _):
        eoff = pl.multiple_of((wid * NCHK + j) * CH * RW, 8)
        ce = pltpu.async_copy(ex_hbm.at[pl.ds(eoff, CH * RW)], exf_v, sem_e)
        pltpu.sync_copy(edges_hbm.at[wid * NCHK + j], ed_ch)
        ci = pltpu.async_copy(inv_hbm.at[ed_ch.at[1]], iv_v, sem_i)
        cx = pltpu.async_copy(x_hbm.at[ed_ch.at[0]], rows_v, sem_x)
        ce.wait()
        ci.wait()
        cx.wait()

        def _edge(e, _):
            av = exf_v[pl.ds(e * RW, RW)] * iv_v[e, pl.ds(0, RW)]
            bh = [av.at[jnp.full((16,), h, jnp.int32)]
                  .get(mode="promise_in_bounds") for h in range(HEADS)]
            for c in range(HID // 16):
                acc = bh[0] * rows_v[e, pl.ds(c * 16, 16)]
                acc = acc + bh[1] * rows_v[e, pl.ds(HID + c * 16, 16)]
                acc = acc + bh[2] * rows_v[e, pl.ds(2 * HID + c * 16, 16)]
                acc = acc + bh[3] * rows_v[e, pl.ds(3 * HID + c * 16, 16)]
                m_v[e, pl.ds(c * 16, 16)] = acc
            return _
        lax.fori_loop(0, CH, _edge, None)

        pltpu.sync_copy(m_v, out_sh.at[ed_ch.at[1]], add=True)
        return _

    lax.fori_loop(0, NCHK, _chunk, None)

    plsc.subcore_barrier()
    pltpu.sync_copy(out_sh.at[pl.ds(sid * NPT, NPT)],
                    out_hbm.at[cid, pl.ds(sid * NPT, NPT)])


_pass2 = functools.partial(
    pl.kernel,
    out_type=jax.ShapeDtypeStruct((NC, NPAD, HID), jnp.float32),
    mesh=_mesh,
    scratch_types=[
        pltpu.VMEM_SHARED((NPAD, HID), jnp.float32),
        pltpu.VMEM((ZR, HID), jnp.float32),
        pltpu.VMEM((2, CH), jnp.int32),
        pltpu.VMEM((CH * RW,), jnp.float32),
        pltpu.VMEM((CH, HID), jnp.float32),
        pltpu.VMEM((CH, HEADS * HID), jnp.float32),
        pltpu.VMEM((CH, HID), jnp.float32),
        pltpu.SemaphoreType.DMA,
        pltpu.SemaphoreType.DMA,
        pltpu.SemaphoreType.DMA,
    ],
)(_pass2_body)


def _amat(a_s, a_d):
    eye = jnp.eye(HEADS, HEADS, dtype=jnp.float32)
    z4 = jnp.zeros((HEADS, HID, HEADS), jnp.float32)
    m = jnp.concatenate(
        [jnp.einsum("hc,hk->hck", a_s, eye), z4,
         jnp.einsum("hc,hk->hck", a_d, eye), z4], axis=-1)
    return m.reshape(HEADS * HID, RW)


def kernel(h_q_raw, h_m_raw, edge_index, Wq, bq, Wm, bm,
           W0, as0, ad0, b0, W1, as1, ad1, b1):
    edges = (edge_index.astype(jnp.int32)
             .reshape(2, NW, NCHK, CH)
             .transpose(1, 2, 0, 3)
             .reshape(NW * NCHK, 2, CH))
    ws = jnp.stack([W0, W1])
    ams = jnp.stack([_amat(as0, ad0), _amat(as1, ad1)])
    bs = jnp.stack([b0, b1])

    h = _proj(h_q_raw, h_m_raw, Wq, bq, Wm, bm)

    def _layer(carry, _x):
        hc, _, li = carry
        x, al = _xal(li, hc, ws, ams)
        ex, den = _pass1(al, edges)
        inv = _inv(den)
        o = _pass2(x, edges, ex, inv)
        raw, h_next = _merge2(li, o[0], o[1], bs)
        return (h_next, raw, li + 1), None

    init = (h, jnp.zeros((NPAD, HID), jnp.float32),
            jnp.zeros((1,), jnp.int32))
    (_, raw_last, _), _ = lax.scan(_layer, init, None, length=2)
    return _split(raw_last)


# speedup vs baseline: 22.6802x; 1.0269x over previous
import functools

import jax
import jax.numpy as jnp
from jax import lax
from jax.experimental import pallas as pl
from jax.experimental.pallas import tpu as pltpu
from jax.experimental.pallas import tpu_sc as plsc

NQ = 5000
NM = 5000
NN = NQ + NM
EE = 320000
DIN = 128
HID = 128
HEADS = 4

NC = 2
NS = 16
NW = NC * NS
EPT = EE // NW
CH = 40
NCHK = EPT // CH
NPAD = 10112
NPT = NPAD // NS
RW = 16

_mesh = plsc.VectorSubcoreMesh(core_axis_name="c", subcore_axis_name="s")


def _f32(*shape):
    return jax.ShapeDtypeStruct(shape, jnp.float32)


def _elu(v):
    return jnp.where(v > 0, v, jnp.exp(jnp.minimum(v, 0.0)) - 1.0)


def _proj_body(hq_ref, hm_ref, wq_ref, bq_ref, wm_ref, bm_ref, h_ref):
    hq = _elu(hq_ref[...] @ wq_ref[...] + bq_ref[...][None, :])
    hm = _elu(hm_ref[...] @ wm_ref[...] + bm_ref[...][None, :])
    pad = jnp.zeros((NPAD - NN, DIN), jnp.float32)
    h_ref[...] = jnp.concatenate([hq, hm, pad], axis=0)


def _proj(hq, hm, wq, bq, wm, bm):
    return pl.pallas_call(_proj_body, out_shape=_f32(NPAD, DIN))(hq, hm, wq, bq, wm, bm)


def _xal_body(li_ref, h_ref, w_ref, a_ref, x_ref, al_ref):
    li = li_ref[0]
    x = jnp.dot(h_ref[...], w_ref[li],
                preferred_element_type=jnp.float32)
    x_ref[...] = x
    al_ref[...] = jnp.dot(x, a_ref[li], preferred_element_type=jnp.float32)


def _xal(li, h, ws, ams):
    blk = NPAD // 8
    return pl.pallas_call(
        _xal_body,
        grid=(NPAD // blk,),
        in_specs=[
            pl.BlockSpec(memory_space=pltpu.SMEM),
            pl.BlockSpec((blk, HID), lambda i: (i, 0)),
            pl.BlockSpec((2, HID, HEADS * HID), lambda i: (0, 0, 0)),
            pl.BlockSpec((2, HEADS * HID, RW), lambda i: (0, 0, 0)),
        ],
        out_specs=[
            pl.BlockSpec((blk, HEADS * HID), lambda i: (i, 0)),
            pl.BlockSpec((blk, RW), lambda i: (i, 0)),
        ],
        out_shape=(_f32(NPAD, HEADS * HID), _f32(NPAD, RW)),
    )(li, h, ws, ams)


def _merge2(li, o0, o1, bs):
    def body(li_ref, o0_ref, o1_ref, b_ref, raw_ref, h_ref):
        v = o0_ref[...] + o1_ref[...] + b_ref[li_ref[0]][None, :]
        raw_ref[...] = v
        h_ref[...] = _elu(v)

    return pl.pallas_call(
        body,
        in_specs=[
            pl.BlockSpec(memory_space=pltpu.SMEM),
            pl.BlockSpec((NPAD, HID), lambda: (0, 0)),
            pl.BlockSpec((NPAD, HID), lambda: (0, 0)),
            pl.BlockSpec((2, HID), lambda: (0, 0)),
        ],
        out_shape=(_f32(NPAD, HID), _f32(NPAD, HID)))(li, o0, o1, bs)


def _split_body(raw_ref, q_ref, m_ref):
    q_ref[...] = raw_ref[...][:NQ]
    m_ref[...] = raw_ref[...][NQ:NN]


def _split(raw):
    return pl.pallas_call(
        _split_body, out_shape=(_f32(NQ, HID), _f32(NM, HID)))(raw)


def _inv_body(dp_ref, inv_ref):
    d = dp_ref[...][0] + dp_ref[...][1]
    i16 = 0.25 / (d + 1e-16)
    inv_ref[...] = jnp.concatenate([i16] * (HID // RW), axis=1)


def _inv(den_parts):
    return pl.pallas_call(_inv_body, out_shape=_f32(NPAD, HID))(den_parts)


ZRD = 8


def _pass1_body(alpha_hbm, edges_hbm, ex_hbm, den_hbm,
                alpha_sh, den_sh, zero_v, src_ch, dst_ch,
                s_rows, d_rows, exf_v, ex2_v, sem_s, sem_d):
    cid = lax.axis_index("c")
    sid = lax.axis_index("s")
    wid = cid * NS + sid

    def _zrow(i, _):
        zero_v[i, pl.ds(0, RW)] = jnp.zeros((RW,), jnp.float32)
        return _
    lax.fori_loop(0, ZRD, _zrow, None)

    def _zden(r, _):
        pltpu.sync_copy(zero_v, den_sh.at[pl.ds(sid * NPT + r * ZRD, ZRD)])
        return _
    lax.fori_loop(0, NPT // ZRD, _zden, None)
    pltpu.sync_copy(alpha_hbm.at[pl.ds(sid * NPT, NPT)],
                    alpha_sh.at[pl.ds(sid * NPT, NPT)])
    plsc.subcore_barrier()
    lt = lax.iota(jnp.int32, 16)
    shfl = jnp.bitwise_and(lt, HEADS * 2 - 1) + HEADS * 2

    def _chunk(j, _):
        soff = pl.multiple_of((wid * NCHK + j) * CH, 8)
        doff = pl.multiple_of(EE + (wid * NCHK + j) * CH, 8)
        ca = pltpu.async_copy(edges_hbm.at[pl.ds(soff, CH)], src_ch, sem_s)
        cb = pltpu.async_copy(edges_hbm.at[pl.ds(doff, CH)], dst_ch, sem_d)
        ca.wait()
        cb.wait()
        cs = pltpu.async_copy(alpha_sh.at[src_ch], s_rows, sem_s)
        cd = pltpu.async_copy(alpha_sh.at[dst_ch], d_rows, sem_d)
        cs.wait()
        cd.wait()

        def _edge(e, _):
            dsh = d_rows[e, pl.ds(0, RW)].at[shfl].get(
                mode="promise_in_bounds")
            al = s_rows[e, pl.ds(0, RW)] + dsh
            al = jnp.maximum(al, 0.2 * al)
            ex = jnp.exp(al)
            exf_v[pl.ds(e * RW, RW)] = ex
            ex2_v[e, pl.ds(0, RW)] = ex
            return _
        lax.fori_loop(0, CH, _edge, None)

        pltpu.sync_copy(ex2_v, den_sh.at[dst_ch], add=True)
        eoff = pl.multiple_of((wid * NCHK + j) * CH * RW, 8)
        pltpu.sync_copy(exf_v, ex_hbm.at[pl.ds(eoff, CH * RW)])
        return _

    lax.fori_loop(0, NCHK, _chunk, None)

    plsc.subcore_barrier()
    pltpu.sync_copy(den_sh.at[pl.ds(sid * NPT, NPT)],
                    den_hbm.at[cid, pl.ds(sid * NPT, NPT)])


_pass1 = functools.partial(
    pl.kernel,
    out_type=(jax.ShapeDtypeStruct((EE * RW,), jnp.float32),
              jax.ShapeDtypeStruct((NC, NPAD, RW), jnp.float32)),
    mesh=_mesh,
    scratch_types=[
        pltpu.VMEM_SHARED((NPAD, RW), jnp.float32),
        pltpu.VMEM_SHARED((NPAD, RW), jnp.float32),
        pltpu.VMEM((ZRD, RW), jnp.float32),
        pltpu.VMEM((CH,), jnp.int32),
        pltpu.VMEM((CH,), jnp.int32),
        pltpu.VMEM((CH, RW), jnp.float32),
        pltpu.VMEM((CH, RW), jnp.float32),
        pltpu.VMEM((CH * RW,), jnp.float32),
        pltpu.VMEM((CH, RW), jnp.float32),
        pltpu.SemaphoreType.DMA,
        pltpu.SemaphoreType.DMA,
    ],
)(_pass1_body)


ZR = 8


def _pass2_body(x_hbm, edges_hbm, ex_hbm, inv_hbm, out_hbm,
                out_sh, zero_v, src_ch, dst_ch,
                exf_v, iv_v, rows_v, m_v, sem_i, sem_x, sem_e):
    cid = lax.axis_index("c")
    sid = lax.axis_index("s")
    wid = cid * NS + sid

    def _zrow(i, _):
        for k in range(HID // 16):
            zero_v[i, pl.ds(k * 16, 16)] = jnp.zeros((16,), jnp.float32)
        return _
    lax.fori_loop(0, ZR, _zrow, None)

    def _zout(r, _):
        pltpu.sync_copy(zero_v, out_sh.at[pl.ds(sid * NPT + r * ZR, ZR)])
        return _
    lax.fori_loop(0, NPT // ZR, _zout, None)

    plsc.subcore_barrier()

    def _chunk(j, _):
        soff = pl.multiple_of((wid * NCHK + j) * CH, 8)
        doff = pl.multiple_of(EE + (wid * NCHK + j) * CH, 8)
        ca = pltpu.async_copy(edges_hbm.at[pl.ds(soff, CH)], src_ch, sem_i)
        cb = pltpu.async_copy(edges_hbm.at[pl.ds(doff, CH)], dst_ch, sem_x)
        eoff = pl.multiple_of((wid * NCHK + j) * CH * RW, 8)
        ce = pltpu.async_copy(ex_hbm.at[pl.ds(eoff, CH * RW)], exf_v, sem_e)
        ca.wait()
        cb.wait()
        ci = pltpu.async_copy(inv_hbm.at[dst_ch], iv_v, sem_i)
        cx = pltpu.async_copy(x_hbm.at[src_ch], rows_v, sem_x)
        ce.wait()
        ci.wait()
        cx.wait()

        def _edge(e, _):
            av = exf_v[pl.ds(e * RW, RW)] * iv_v[e, pl.ds(0, RW)]
            bh = [av.at[jnp.full((16,), h, jnp.int32)]
                  .get(mode="promise_in_bounds") for h in range(HEADS)]
            for c in range(HID // 16):
                acc = bh[0] * rows_v[e, pl.ds(c * 16, 16)]
                acc = acc + bh[1] * rows_v[e, pl.ds(HID + c * 16, 16)]
                acc = acc + bh[2] * rows_v[e, pl.ds(2 * HID + c * 16, 16)]
                acc = acc + bh[3] * rows_v[e, pl.ds(3 * HID + c * 16, 16)]
                m_v[e, pl.ds(c * 16, 16)] = acc
            return _
        lax.fori_loop(0, CH, _edge, None)

        pltpu.sync_copy(m_v, out_sh.at[dst_ch], add=True)
        return _

    lax.fori_loop(0, NCHK, _chunk, None)

    plsc.subcore_barrier()
    pltpu.sync_copy(out_sh.at[pl.ds(sid * NPT, NPT)],
                    out_hbm.at[cid, pl.ds(sid * NPT, NPT)])


_pass2 = functools.partial(
    pl.kernel,
    out_type=jax.ShapeDtypeStruct((NC, NPAD, HID), jnp.float32),
    mesh=_mesh,
    scratch_types=[
        pltpu.VMEM_SHARED((NPAD, HID), jnp.float32),
        pltpu.VMEM((ZR, HID), jnp.float32),
        pltpu.VMEM((CH,), jnp.int32),
        pltpu.VMEM((CH,), jnp.int32),
        pltpu.VMEM((CH * RW,), jnp.float32),
        pltpu.VMEM((CH, HID), jnp.float32),
        pltpu.VMEM((CH, HEADS * HID), jnp.float32),
        pltpu.VMEM((CH, HID), jnp.float32),
        pltpu.SemaphoreType.DMA,
        pltpu.SemaphoreType.DMA,
        pltpu.SemaphoreType.DMA,
    ],
)(_pass2_body)


def _amat(a_s, a_d):
    eye = jnp.eye(HEADS, HEADS, dtype=jnp.float32)
    z4 = jnp.zeros((HEADS, HID, HEADS), jnp.float32)
    m = jnp.concatenate(
        [jnp.einsum("hc,hk->hck", a_s, eye), z4,
         jnp.einsum("hc,hk->hck", a_d, eye), z4], axis=-1)
    return m.reshape(HEADS * HID, RW)


def kernel(h_q_raw, h_m_raw, edge_index, Wq, bq, Wm, bm,
           W0, as0, ad0, b0, W1, as1, ad1, b1):
    edges = edge_index.astype(jnp.int32).reshape(2 * EE)
    ws = jnp.stack([W0, W1])
    ams = jnp.stack([_amat(as0, ad0), _amat(as1, ad1)])
    bs = jnp.stack([b0, b1])

    h = _proj(h_q_raw, h_m_raw, Wq, bq, Wm, bm)

    def _layer(carry, _x):
        hc, _, li = carry
        x, al = _xal(li, hc, ws, ams)
        ex, den = _pass1(al, edges)
        inv = _inv(den)
        o = _pass2(x, edges, ex, inv)
        raw, h_next = _merge2(li, o[0], o[1], bs)
        return (h_next, raw, li + 1), None

    init = (h, jnp.zeros((NPAD, HID), jnp.float32),
            jnp.zeros((1,), jnp.int32))
    (_, raw_last, _), _ = lax.scan(_layer, init, None, length=2)
    return _split(raw_last)
